# baseline (device time: 21195 ns/iter reference)
import jax
import jax.numpy as jnp
from jax import lax
from jax.experimental import pallas as pl
from jax.experimental.pallas import tpu as pltpu


def kernel(Q, K, V):
    b, q, h, d = Q.shape
    kv = K.shape[1]
    scale = d ** -0.5

    Kt = jnp.transpose(K, (0, 2, 3, 1))
    Vt = jnp.transpose(V, (0, 2, 3, 1))
    eye8 = jnp.eye(h, dtype=jnp.float32)
    Qbd = (Q[:, 0, :, None, :] * (eye8 * scale)[None, :, :, None]).reshape(
        b, h, h * d
    )

    def body(
        qbd_ref, kt_hbm, vt_hbm, out_ref,
        kbuf, vbuf, cp_sems, comm, comm_sems,
    ):
        my_x = lax.axis_index("x")
        my_y = lax.axis_index("y")
        nbr = (my_x, 1 - my_y)

        barrier = pltpu.get_barrier_semaphore()
        pl.semaphore_signal(
            barrier, inc=1, device_id=nbr, device_id_type=pl.DeviceIdType.MESH
        )

        def start(bi):
            slot = bi % 2
            kc = pltpu.make_async_copy(
                kt_hbm.at[bi], kbuf.at[slot], cp_sems.at[slot, 0]
            )
            vc = pltpu.make_async_copy(
                vt_hbm.at[bi], vbuf.at[slot], cp_sems.at[slot, 1]
            )
            kc.start()
            vc.start()
            return kc, vc

        pend = {0: start(0), 1: start(1)}

        ii = lax.broadcasted_iota(jnp.int32, (h, h), 0)
        jj = lax.broadcasted_iota(jnp.int32, (h, h), 1)
        eyem = (ii == jj).astype(jnp.float32)

        o_list = []
        l_list = []
        for bi in range(b):
            kc, vc = pend.pop(bi)
            kc.wait()
            vc.wait()
            slot = bi % 2
            k2 = kbuf[slot].reshape(h * d, kv)
            s_b = lax.dot_general(
                qbd_ref[bi], k2,
                dimension_numbers=(((1,), (0,)), ((), ())),
                preferred_element_type=jnp.float32,
            )
            p_b = jnp.exp(s_b)
            l_b = jnp.sum(p_b, axis=1)
            v2 = vbuf[slot].reshape(h * d, kv)
            gv = lax.dot_general(
                v2, p_b,
                dimension_numbers=(((1,), (1,)), ((), ())),
                preferred_element_type=jnp.float32,
            )
            o_b = jnp.sum(
                gv.reshape(h, d, h) * eyem[:, None, :], axis=-1
            )
            o_list.append(o_b)
            l_list.append(l_b)
            if bi + 2 < b:
                pend[bi + 2] = start(bi + 2)
        o = jnp.stack(o_list)
        l = jnp.stack(l_list)

        comm[0, :b] = o
        comm[0, b, :, :h] = l

        pl.semaphore_wait(barrier, 1)
        rdma = pltpu.make_async_remote_copy(
            src_ref=comm.at[0],
            dst_ref=comm.at[1],
            send_sem=comm_sems.at[0],
            recv_sem=comm_sems.at[1],
            device_id=nbr,
            device_id_type=pl.DeviceIdType.MESH,
        )
        rdma.start()
        rdma.wait()

        denom = l + comm[1, b, :, :h]
        o_full = (o + comm[1, :b]) / denom[..., None]
        out_ref[...] = o_full[:, None, :, :]

    return pl.pallas_call(
        body,
        out_shape=jax.ShapeDtypeStruct((b, q, h, d), jnp.float32),
        in_specs=[
            pl.BlockSpec(memory_space=pltpu.VMEM),
            pl.BlockSpec(memory_space=pl.ANY),
            pl.BlockSpec(memory_space=pl.ANY),
        ],
        out_specs=pl.BlockSpec(memory_space=pltpu.VMEM),
        scratch_shapes=[
            pltpu.VMEM((2, h, d, kv), jnp.float32),
            pltpu.VMEM((2, h, d, kv), jnp.float32),
            pltpu.SemaphoreType.DMA((2, 2)),
            pltpu.VMEM((2, b + 1, h, d), jnp.float32),
            pltpu.SemaphoreType.DMA((2,)),
        ],
        compiler_params=pltpu.CompilerParams(collective_id=0),
    )(Qbd, Kt, Vt)


# device time: 17562 ns/iter; 1.2069x vs baseline; 1.2069x over previous
import jax
import jax.numpy as jnp
from jax import lax
from jax.experimental import pallas as pl
from jax.experimental.pallas import tpu as pltpu


def kernel(Q, K, V):
    b, q, h, d = Q.shape
    kv = K.shape[1]
    scale = d ** -0.5

    Kt = jnp.transpose(K, (0, 2, 3, 1))
    Vt = jnp.transpose(V, (0, 2, 3, 1))
    eye8 = jnp.eye(h, dtype=jnp.float32)
    Qbd = (Q[:, 0, :, None, :] * (eye8 * scale)[None, :, :, None]).reshape(
        b, h, h * d
    )

    def body(
        qbd_ref, kt_hbm, vt_hbm, out_ref,
        kbuf, vbuf, cp_sems, comm, comm_sems,
    ):
        my_x = lax.axis_index("x")
        my_y = lax.axis_index("y")
        nbr = (my_x, 1 - my_y)

        barrier = pltpu.get_barrier_semaphore()
        pl.semaphore_signal(
            barrier, inc=1, device_id=nbr, device_id_type=pl.DeviceIdType.MESH
        )

        hb = b // 2
        copies = []
        for half in range(2):
            sl = slice(half * hb, (half + 1) * hb)
            kc = pltpu.make_async_copy(
                kt_hbm.at[sl], kbuf.at[sl], cp_sems.at[half, 0]
            )
            vc = pltpu.make_async_copy(
                vt_hbm.at[sl], vbuf.at[sl], cp_sems.at[half, 1]
            )
            copies.append((kc, vc))
        copies[0][0].start()
        copies[0][1].start()
        copies[1][0].start()
        copies[1][1].start()

        ii = lax.broadcasted_iota(jnp.int32, (h, h), 0)
        jj = lax.broadcasted_iota(jnp.int32, (h, h), 1)
        eyem = (ii == jj).astype(jnp.float32)

        o_list = []
        l_list = []
        for bi in range(b):
            if bi % hb == 0:
                kc, vc = copies[bi // hb]
                kc.wait()
                vc.wait()
            k2 = kbuf[bi].reshape(h * d, kv)
            s_b = lax.dot_general(
                qbd_ref[bi], k2,
                dimension_numbers=(((1,), (0,)), ((), ())),
                preferred_element_type=jnp.float32,
            )
            p_b = jnp.exp(s_b)
            l_b = jnp.sum(p_b, axis=1)
            v2 = vbuf[bi].reshape(h * d, kv)
            gv = lax.dot_general(
                v2, p_b,
                dimension_numbers=(((1,), (1,)), ((), ())),
                preferred_element_type=jnp.float32,
            )
            o_b = jnp.sum(
                gv.reshape(h, d, h) * eyem[:, None, :], axis=-1
            )
            o_list.append(o_b)
            l_list.append(l_b)
        o = jnp.stack(o_list)
        l = jnp.stack(l_list)

        comm[0, :b] = o
        comm[0, b, :, :h] = l

        pl.semaphore_wait(barrier, 1)
        rdma = pltpu.make_async_remote_copy(
            src_ref=comm.at[0],
            dst_ref=comm.at[1],
            send_sem=comm_sems.at[0],
            recv_sem=comm_sems.at[1],
            device_id=nbr,
            device_id_type=pl.DeviceIdType.MESH,
        )
        rdma.start()
        rdma.wait()

        denom = l + comm[1, b, :, :h]
        o_full = (o + comm[1, :b]) / denom[..., None]
        out_ref[...] = o_full[:, None, :, :]

    return pl.pallas_call(
        body,
        out_shape=jax.ShapeDtypeStruct((b, q, h, d), jnp.float32),
        in_specs=[
            pl.BlockSpec(memory_space=pltpu.VMEM),
            pl.BlockSpec(memory_space=pl.ANY),
            pl.BlockSpec(memory_space=pl.ANY),
        ],
        out_specs=pl.BlockSpec(memory_space=pltpu.VMEM),
        scratch_shapes=[
            pltpu.VMEM((b, h, d, kv), jnp.float32),
            pltpu.VMEM((b, h, d, kv), jnp.float32),
            pltpu.SemaphoreType.DMA((2, 2)),
            pltpu.VMEM((2, b + 1, h, d), jnp.float32),
            pltpu.SemaphoreType.DMA((2,)),
        ],
        compiler_params=pltpu.CompilerParams(
            collective_id=0,
            vmem_limit_bytes=120 * 1024 * 1024,
        ),
    )(Qbd, Kt, Vt)


# device time: 15042 ns/iter; 1.4091x vs baseline; 1.1675x over previous
import jax
import jax.numpy as jnp
from jax import lax
from jax.experimental import pallas as pl
from jax.experimental.pallas import tpu as pltpu

N_CHUNKS = 4


def kernel(Q, K, V):
    b, q, h, d = Q.shape
    kv = K.shape[1]
    scale = d ** -0.5

    Kt = jnp.transpose(K, (0, 2, 3, 1))
    Vt = jnp.transpose(V, (0, 2, 3, 1))

    cb = b // N_CHUNKS

    def body(q_ref, kt_hbm, vt_hbm, out_ref, kbuf, vbuf, cp_sems, comm, comm_sems):
        my_x = lax.axis_index("x")
        my_y = lax.axis_index("y")
        nbr = (my_x, 1 - my_y)

        barrier = pltpu.get_barrier_semaphore()
        pl.semaphore_signal(
            barrier, inc=1, device_id=nbr, device_id_type=pl.DeviceIdType.MESH
        )

        copies = []
        for c in range(N_CHUNKS):
            sl = slice(c * cb, (c + 1) * cb)
            kc = pltpu.make_async_copy(
                kt_hbm.at[sl], kbuf.at[sl], cp_sems.at[c, 0]
            )
            vc = pltpu.make_async_copy(
                vt_hbm.at[sl], vbuf.at[sl], cp_sems.at[c, 1]
            )
            copies.append((kc, vc))
        for kc, vc in copies:
            kc.start()
            vc.start()

        ii = lax.broadcasted_iota(jnp.int32, (h, h), 0)
        jj = lax.broadcasted_iota(jnp.int32, (h, h), 1)
        eyem = (ii == jj).astype(jnp.float32)
        colh = lax.broadcasted_iota(jnp.int32, (h, h * d), 1) // d
        rowh = lax.broadcasted_iota(jnp.int32, (h, h * d), 0)
        bdmask = jnp.where(colh == rowh, scale, 0.0)

        o_list = []
        l_list = []
        for bi in range(b):
            if bi % cb == 0:
                kc, vc = copies[bi // cb]
                kc.wait()
                vc.wait()
            qb = q_ref[bi, 0]
            qbd = (
                jnp.broadcast_to(qb[:, None, :], (h, h, d)).reshape(h, h * d)
                * bdmask
            )
            k2 = kbuf[bi].reshape(h * d, kv)
            s_b = lax.dot_general(
                qbd, k2,
                dimension_numbers=(((1,), (0,)), ((), ())),
                preferred_element_type=jnp.float32,
            )
            p_b = jnp.exp(s_b)
            l_b = jnp.sum(p_b, axis=1)
            v2 = vbuf[bi].reshape(h * d, kv)
            gv = lax.dot_general(
                v2, p_b,
                dimension_numbers=(((1,), (1,)), ((), ())),
                preferred_element_type=jnp.float32,
            )
            o_b = jnp.sum(
                gv.reshape(h, d, h) * eyem[:, None, :], axis=-1
            )
            o_list.append(o_b)
            l_list.append(l_b)
        o = jnp.stack(o_list)
        l = jnp.stack(l_list)

        comm[0, :b] = o
        comm[0, b, :, :h] = l

        pl.semaphore_wait(barrier, 1)
        rdma = pltpu.make_async_remote_copy(
            src_ref=comm.at[0],
            dst_ref=comm.at[1],
            send_sem=comm_sems.at[0],
            recv_sem=comm_sems.at[1],
            device_id=nbr,
            device_id_type=pl.DeviceIdType.MESH,
        )
        rdma.start()
        rdma.wait()

        denom = l + comm[1, b, :, :h]
        o_full = (o + comm[1, :b]) / denom[..., None]
        out_ref[...] = o_full[:, None, :, :]

    return pl.pallas_call(
        body,
        out_shape=jax.ShapeDtypeStruct((b, q, h, d), jnp.float32),
        in_specs=[
            pl.BlockSpec(memory_space=pltpu.VMEM),
            pl.BlockSpec(memory_space=pl.ANY),
            pl.BlockSpec(memory_space=pl.ANY),
        ],
        out_specs=pl.BlockSpec(memory_space=pltpu.VMEM),
        scratch_shapes=[
            pltpu.VMEM((b, h, d, kv), jnp.float32),
            pltpu.VMEM((b, h, d, kv), jnp.float32),
            pltpu.SemaphoreType.DMA((N_CHUNKS, 2)),
            pltpu.VMEM((2, b + 1, h, d), jnp.float32),
            pltpu.SemaphoreType.DMA((2,)),
        ],
        compiler_params=pltpu.CompilerParams(
            collective_id=0,
            vmem_limit_bytes=120 * 1024 * 1024,
        ),
    )(Q, Kt, Vt)


# device time: 13945 ns/iter; 1.5199x vs baseline; 1.0787x over previous
import jax
import jax.numpy as jnp
from jax import lax
from jax.experimental import pallas as pl
from jax.experimental.pallas import tpu as pltpu

CHUNKS = ((0, 1), (1, 2), (2, 4), (4, 8))


def kernel(Q, K, V):
    b, q, h, d = Q.shape
    kv = K.shape[1]
    scale = d ** -0.5

    Kt = jnp.transpose(K, (0, 2, 3, 1))
    Vt = jnp.transpose(V, (0, 2, 3, 1))

    def body(q_ref, kt_hbm, vt_hbm, out_ref, kbuf, vbuf, cp_sems, comm, comm_sems):
        my_x = lax.axis_index("x")
        my_y = lax.axis_index("y")
        nbr = (my_x, 1 - my_y)

        barrier = pltpu.get_barrier_semaphore()
        pl.semaphore_signal(
            barrier, inc=1, device_id=nbr, device_id_type=pl.DeviceIdType.MESH
        )

        copies = []
        for c, (lo, hi) in enumerate(CHUNKS):
            sl = slice(lo, hi)
            kc = pltpu.make_async_copy(
                kt_hbm.at[sl], kbuf.at[sl], cp_sems.at[c, 0]
            )
            vc = pltpu.make_async_copy(
                vt_hbm.at[sl], vbuf.at[sl], cp_sems.at[c, 1]
            )
            copies.append((kc, vc))
        for kc, vc in copies:
            kc.start()
            vc.start()

        ii = lax.broadcasted_iota(jnp.int32, (h, h), 0)
        jj = lax.broadcasted_iota(jnp.int32, (h, h), 1)
        eyem = (ii == jj).astype(jnp.float32)
        colh = lax.broadcasted_iota(jnp.int32, (h, h * d), 1) // d
        rowh = lax.broadcasted_iota(jnp.int32, (h, h * d), 0)
        bdmask = jnp.where(colh == rowh, scale, 0.0)

        for ci, (lo, hi) in enumerate(CHUNKS):
            kc, vc = copies[ci]
            kc.wait()
            for bi in range(lo, hi):
                qb = q_ref[bi, 0]
                qbd = (
                    jnp.broadcast_to(qb[:, None, :], (h, h, d)).reshape(h, h * d)
                    * bdmask
                )
                k2 = kbuf[bi].reshape(h * d, kv)
                s_b = lax.dot_general(
                    qbd, k2,
                    dimension_numbers=(((1,), (0,)), ((), ())),
                    preferred_element_type=jnp.float32,
                )
                p_b = jnp.exp(s_b)
                l_b = jnp.sum(p_b, axis=1)
                if bi == lo:
                    vc.wait()
                v2 = vbuf[bi].reshape(h * d, kv)
                gv = lax.dot_general(
                    v2, p_b,
                    dimension_numbers=(((1,), (1,)), ((), ())),
                    preferred_element_type=jnp.float32,
                )
                o_b = jnp.sum(
                    gv.reshape(h, d, h) * eyem[:, None, :], axis=-1
                )
                comm[0, bi] = o_b
                comm[0, b, bi : bi + 1, :h] = l_b[None, :]

        pl.semaphore_wait(barrier, 1)
        rdma = pltpu.make_async_remote_copy(
            src_ref=comm.at[0],
            dst_ref=comm.at[1],
            send_sem=comm_sems.at[0],
            recv_sem=comm_sems.at[1],
            device_id=nbr,
            device_id_type=pl.DeviceIdType.MESH,
        )
        rdma.start()
        rdma.wait()

        denom = comm[0, b, :, :h] + comm[1, b, :, :h]
        o_full = (comm[0, :b] + comm[1, :b]) / denom[..., None]
        out_ref[...] = o_full[:, None, :, :]

    return pl.pallas_call(
        body,
        out_shape=jax.ShapeDtypeStruct((b, q, h, d), jnp.float32),
        in_specs=[
            pl.BlockSpec(memory_space=pltpu.VMEM),
            pl.BlockSpec(memory_space=pl.ANY),
            pl.BlockSpec(memory_space=pl.ANY),
        ],
        out_specs=pl.BlockSpec(memory_space=pltpu.VMEM),
        scratch_shapes=[
            pltpu.VMEM((b, h, d, kv), jnp.float32),
            pltpu.VMEM((b, h, d, kv), jnp.float32),
            pltpu.SemaphoreType.DMA((len(CHUNKS), 2)),
            pltpu.VMEM((2, b + 1, h, d), jnp.float32),
            pltpu.SemaphoreType.DMA((2,)),
        ],
        compiler_params=pltpu.CompilerParams(
            collective_id=0,
            vmem_limit_bytes=120 * 1024 * 1024,
        ),
    )(Q, Kt, Vt)
